# baseline (speedup 1.0000x reference)
import functools

import jax
import jax.numpy as jnp
from jax import lax
from jax.experimental import pallas as pl
from jax.experimental.pallas import tpu as pltpu
from jax.experimental.pallas import tpu_sc as plsc

NC = 2
NS = 16
L = 16

M, D, B = 100000, 128, 16384
NBLK = 20
R = M // NBLK
BPC = NBLK // NC
RPT = 312
RLAST = R - (NS - 1) * RPT
BPT = B // NS
NV = BPT // L
C = 32
TRASH = R


def _sc_body(in_hbm, idx_hbm, val_hbm, out_hbm,
             my_idx, loc_buf, pos_buf,
             loc0, pos0, loc1, pos1, vbuf0, vbuf1,
             blk0, blk1, g0, g1, si0, si1, so0, so1):
    c = lax.axis_index("c")
    s = lax.axis_index("s")

    blks = [blk0, blk1]
    sins = [si0, si1]
    souts = [so0, so1]

    def tile_slices(i):
        base = (NC * i + c) * R
        return base

    def copy_in(i, buf, sem, start):
        base = tile_slices(i)

        @pl.when(s < NS - 1)
        def _():
            cp = pltpu.make_async_copy(
                in_hbm.at[pl.ds(base + s * RPT, RPT)],
                buf.at[pl.ds(s * RPT, RPT)], sem)
            cp.start() if start else cp.wait()

        @pl.when(s == NS - 1)
        def _():
            cp = pltpu.make_async_copy(
                in_hbm.at[pl.ds(base + (NS - 1) * RPT, RLAST)],
                buf.at[pl.ds((NS - 1) * RPT, RLAST)], sem)
            cp.start() if start else cp.wait()

    def copy_out(i, buf, sem, start):
        base = tile_slices(i)

        @pl.when(s < NS - 1)
        def _():
            cp = pltpu.make_async_copy(
                buf.at[pl.ds(s * RPT, RPT)],
                out_hbm.at[pl.ds(base + s * RPT, RPT)], sem)
            cp.start() if start else cp.wait()

        @pl.when(s == NS - 1)
        def _():
            cp = pltpu.make_async_copy(
                buf.at[pl.ds((NS - 1) * RPT, RLAST)],
                out_hbm.at[pl.ds(base + (NS - 1) * RPT, RLAST)], sem)
            cp.start() if start else cp.wait()

    pltpu.sync_copy(idx_hbm.at[pl.ds(s * BPT, BPT)], my_idx)

    copy_in(0, blks[0], sins[0], True)

    for i in range(BPC):
        p = i % 2
        blk = blks[p]
        base = tile_slices(i)

        if i + 1 < BPC:
            if i >= 1:
                copy_out(i - 1, blks[1 - p], souts[1 - p], False)
            copy_in(i + 1, blks[1 - p], sins[1 - p], True)

        with jax.named_scope("cin_wait"):
            copy_in(i, blk, sins[p], False)
            plsc.subcore_barrier()

        def cbody(j, cnt):
            iv = my_idx[pl.ds(j * L, L)]
            basev = jnp.full((L,), base, jnp.int32)
            limv = jnp.full((L,), base + R, jnp.int32)
            m = (iv >= basev) & (iv < limv)
            loc = iv - basev
            pos = jnp.full((L,), s * BPT + j * L, jnp.int32) + lax.iota(jnp.int32, L)
            mi = jnp.where(m, 1, 0).astype(jnp.int32)
            pc = plsc.cumsum(mi)
            dest = jnp.full((L,), cnt, jnp.int32) + pc - 1
            plsc.store_scatter(loc_buf, [dest], loc, mask=m)
            plsc.store_scatter(pos_buf, [dest], pos, mask=m)
            return cnt + jnp.sum(mi)

        with jax.named_scope("compact"):
            cnt = lax.fori_loop(0, NV, cbody, jnp.int32(0))
        nch = (cnt + (C - 1)) // C

        def build(ch, lref, pref):
            off = ch * C
            cntv = jnp.full((L,), cnt, jnp.int32)
            for v in range(C // L):
                lane = jnp.full((L,), off + v * L, jnp.int32) + lax.iota(jnp.int32, L)
                valid = lane < cntv
                lv = loc_buf[pl.ds(off + v * L, L)]
                pv = pos_buf[pl.ds(off + v * L, L)]
                lref[pl.ds(v * L, L)] = jnp.where(valid, lv, TRASH)
                pref[pl.ds(v * L, L)] = jnp.where(valid, pv, 0)

        def chunk_step(ch, lA, pA, vA, sA, lB, pB, vB, sB):
            pltpu.make_async_copy(val_hbm.at[pA], vA, sA).wait()

            @pl.when(ch + 1 < nch)
            def _():
                build(ch + 1, lB, pB)
                pltpu.make_async_copy(val_hbm.at[pB], vB, sB).start()

            pltpu.sync_copy(vA, blk.at[lA], add=True)

        @pl.when(nch > 0)
        def _():
            build(0, loc0, pos0)
            pltpu.make_async_copy(val_hbm.at[pos0], vbuf0, g0).start()

        def chbody(ch, carry):
            even = lax.rem(ch, 2) == 0

            @pl.when(even)
            def _():
                chunk_step(ch, loc0, pos0, vbuf0, g0, loc1, pos1, vbuf1, g1)

            @pl.when(jnp.logical_not(even))
            def _():
                chunk_step(ch, loc1, pos1, vbuf1, g1, loc0, pos0, vbuf0, g0)

            return carry

        with jax.named_scope("chunks"):
            lax.fori_loop(0, nch, chbody, jnp.int32(0))
        with jax.named_scope("bar2"):
            plsc.subcore_barrier()

        copy_out(i, blk, souts[p], True)

    copy_out(BPC - 2, blks[(BPC - 2) % 2], souts[(BPC - 2) % 2], False)
    copy_out(BPC - 1, blks[(BPC - 1) % 2], souts[(BPC - 1) % 2], False)


@jax.jit
def _scatter_add(input, idx32, value):
    kfn = functools.partial(
        pl.kernel,
        mesh=plsc.VectorSubcoreMesh(core_axis_name="c", subcore_axis_name="s"),
        out_type=jax.ShapeDtypeStruct((M, D), jnp.float32),
        scratch_types=[
            pltpu.VMEM((BPT,), jnp.int32),
            pltpu.VMEM((BPT + 2 * L,), jnp.int32),
            pltpu.VMEM((BPT + 2 * L,), jnp.int32),
            pltpu.VMEM((C,), jnp.int32),
            pltpu.VMEM((C,), jnp.int32),
            pltpu.VMEM((C,), jnp.int32),
            pltpu.VMEM((C,), jnp.int32),
            pltpu.VMEM((C, D), jnp.float32),
            pltpu.VMEM((C, D), jnp.float32),
            pltpu.VMEM_SHARED((R + L, D), jnp.float32),
            pltpu.VMEM_SHARED((R + L, D), jnp.float32),
            pltpu.SemaphoreType.DMA,
            pltpu.SemaphoreType.DMA,
            pltpu.SemaphoreType.DMA,
            pltpu.SemaphoreType.DMA,
            pltpu.SemaphoreType.DMA,
            pltpu.SemaphoreType.DMA,
        ],
        compiler_params=pltpu.CompilerParams(needs_layout_passes=False),
    )(_sc_body)
    return kfn(input, idx32, value)


def kernel(input, index, value):
    assert input.shape == (M, D) and value.shape == (B, D)
    return _scatter_add(input, index.astype(jnp.int32), value)


# speedup vs baseline: 1.3832x; 1.3832x over previous
import functools

import jax
import jax.numpy as jnp
from jax import lax
from jax.experimental import pallas as pl
from jax.experimental.pallas import tpu as pltpu
from jax.experimental.pallas import tpu_sc as plsc

NC = 2
NS = 16
L = 16

M, D, B = 100000, 128, 16384
NBLK = 20
R = M // NBLK
BPC = NBLK // NC
RPT = 312
RLAST = R - (NS - 1) * RPT
BPT = B // NS
NV = BPT // L
TRASH = R

SZS = (128, 64, 32, 16)
REG = {128: 0, 64: 128, 32: 192, 16: 224}
PASS_ROWS = 128


def _sc_body(in_hbm, idx_hbm, val_hbm, out_hbm,
             my_idx, loc_buf, pos_buf,
             loc128, loc64, loc32, loc16, vbuf,
             blk0, blk1, gg, gs, si0, si1, so0, so1):
    c = lax.axis_index("c")
    s = lax.axis_index("s")

    blks = [blk0, blk1]
    sins = [si0, si1]
    souts = [so0, so1]
    locs = {128: loc128, 64: loc64, 32: loc32, 16: loc16}

    def blk_base(i):
        return (NC * i + c) * R

    def copy_in(i, buf, sem, start):
        base = blk_base(i)

        @pl.when(s < NS - 1)
        def _():
            cp = pltpu.make_async_copy(
                in_hbm.at[pl.ds(base + s * RPT, RPT)],
                buf.at[pl.ds(s * RPT, RPT)], sem)
            cp.start() if start else cp.wait()

        @pl.when(s == NS - 1)
        def _():
            cp = pltpu.make_async_copy(
                in_hbm.at[pl.ds(base + (NS - 1) * RPT, RLAST)],
                buf.at[pl.ds((NS - 1) * RPT, RLAST)], sem)
            cp.start() if start else cp.wait()

    def copy_out(i, buf, sem, start):
        base = blk_base(i)

        @pl.when(s < NS - 1)
        def _():
            cp = pltpu.make_async_copy(
                buf.at[pl.ds(s * RPT, RPT)],
                out_hbm.at[pl.ds(base + s * RPT, RPT)], sem)
            cp.start() if start else cp.wait()

        @pl.when(s == NS - 1)
        def _():
            cp = pltpu.make_async_copy(
                buf.at[pl.ds((NS - 1) * RPT, RLAST)],
                out_hbm.at[pl.ds(base + (NS - 1) * RPT, RLAST)], sem)
            cp.start() if start else cp.wait()

    pltpu.sync_copy(idx_hbm.at[pl.ds(s * BPT, BPT)], my_idx)

    copy_in(0, blks[0], sins[0], True)

    for i in range(BPC):
        p = i % 2
        blk = blks[p]
        base = blk_base(i)

        if i + 1 < BPC:
            if i >= 1:
                copy_out(i - 1, blks[1 - p], souts[1 - p], False)
            copy_in(i + 1, blks[1 - p], sins[1 - p], True)

        copy_in(i, blk, sins[p], False)
        plsc.subcore_barrier()

        def cbody(j, cnt):
            iv = my_idx[pl.ds(j * L, L)]
            basev = jnp.full((L,), base, jnp.int32)
            limv = jnp.full((L,), base + R, jnp.int32)
            m = (iv >= basev) & (iv < limv)
            loc = iv - basev
            pos = jnp.full((L,), s * BPT + j * L, jnp.int32) + lax.iota(jnp.int32, L)
            mi = jnp.where(m, 1, 0).astype(jnp.int32)
            pc = plsc.cumsum(mi)
            dest = jnp.full((L,), cnt, jnp.int32) + pc - 1
            plsc.store_scatter(loc_buf, [dest], loc, mask=m)
            plsc.store_scatter(pos_buf, [dest], pos, mask=m)
            return cnt + jnp.sum(mi)

        cnt = lax.fori_loop(0, NV, cbody, jnp.int32(0))

        vstart = (cnt // L) * L
        lane = jnp.full((L,), vstart, jnp.int32) + lax.iota(jnp.int32, L)
        valid = lane < jnp.full((L,), cnt, jnp.int32)
        lv = loc_buf[pl.ds(vstart, L)]
        pv = pos_buf[pl.ds(vstart, L)]
        loc_buf[pl.ds(vstart, L)] = jnp.where(valid, lv, TRASH)
        pos_buf[pl.ds(vstart, L)] = jnp.where(valid, pv, 0)

        units = (cnt + L - 1) // L
        npass = (units + (PASS_ROWS // L - 1)) // (PASS_ROWS // L)

        def pass_body(ps, carry):
            start_u = ps * (PASS_ROWS // L)
            u = jnp.minimum(units - start_u, PASS_ROWS // L)

            offs = {}
            cur = start_u * L
            for S in SZS:
                offs[S] = cur
                bit = (u // (S // L)) % 2 if S != PASS_ROWS else (u // (S // L))
                cur = cur + bit * S

            conds = {
                128: u >= 8,
                64: (u // 4) % 2 == 1,
                32: (u // 2) % 2 == 1,
                16: u % 2 == 1,
            }

            for S in SZS:
                @pl.when(conds[S])
                def _(S=S):
                    pltpu.async_copy(
                        val_hbm.at[pos_buf.at[pl.ds(offs[S], S)]],
                        vbuf.at[pl.ds(REG[S], S)], gg)
                    for t in range(S // L):
                        locs[S][pl.ds(t * L, L)] = loc_buf[pl.ds(offs[S] + t * L, L)]

            for S in SZS:
                @pl.when(conds[S])
                def _(S=S):
                    pltpu.make_async_copy(
                        val_hbm.at[pos_buf.at[pl.ds(offs[S], S)]],
                        vbuf.at[pl.ds(REG[S], S)], gg).wait()

            for S in SZS:
                @pl.when(conds[S])
                def _(S=S):
                    pltpu.async_copy(
                        vbuf.at[pl.ds(REG[S], S)],
                        blk.at[locs[S]], gs, add=True)

            for S in SZS:
                @pl.when(conds[S])
                def _(S=S):
                    pltpu.make_async_copy(
                        vbuf.at[pl.ds(REG[S], S)],
                        blk.at[locs[S]], gs).wait()

            return carry

        with jax.named_scope("chunks"):
            lax.fori_loop(0, npass, pass_body, jnp.int32(0))
        with jax.named_scope("bar2"):
            plsc.subcore_barrier()

        copy_out(i, blk, souts[p], True)

    copy_out(BPC - 2, blks[(BPC - 2) % 2], souts[(BPC - 2) % 2], False)
    copy_out(BPC - 1, blks[(BPC - 1) % 2], souts[(BPC - 1) % 2], False)


@jax.jit
def _scatter_add(input, idx32, value):
    kfn = functools.partial(
        pl.kernel,
        mesh=plsc.VectorSubcoreMesh(core_axis_name="c", subcore_axis_name="s"),
        out_type=jax.ShapeDtypeStruct((M, D), jnp.float32),
        scratch_types=[
            pltpu.VMEM((BPT,), jnp.int32),
            pltpu.VMEM((BPT + 2 * L,), jnp.int32),
            pltpu.VMEM((BPT + 2 * L,), jnp.int32),
            pltpu.VMEM((128,), jnp.int32),
            pltpu.VMEM((64,), jnp.int32),
            pltpu.VMEM((32,), jnp.int32),
            pltpu.VMEM((16,), jnp.int32),
            pltpu.VMEM((256, D), jnp.float32),
            pltpu.VMEM_SHARED((R + L, D), jnp.float32),
            pltpu.VMEM_SHARED((R + L, D), jnp.float32),
            pltpu.SemaphoreType.DMA,
            pltpu.SemaphoreType.DMA,
            pltpu.SemaphoreType.DMA,
            pltpu.SemaphoreType.DMA,
            pltpu.SemaphoreType.DMA,
            pltpu.SemaphoreType.DMA,
        ],
        compiler_params=pltpu.CompilerParams(needs_layout_passes=False),
    )(_sc_body)
    return kfn(input, idx32, value)


def kernel(input, index, value):
    assert input.shape == (M, D) and value.shape == (B, D)
    return _scatter_add(input, index.astype(jnp.int32), value)


# speedup vs baseline: 1.3989x; 1.0114x over previous
import functools

import jax
import jax.numpy as jnp
from jax import lax
from jax.experimental import pallas as pl
from jax.experimental.pallas import tpu as pltpu
from jax.experimental.pallas import tpu_sc as plsc

NC = 2
NS = 16
L = 16

M, D, B = 100000, 128, 16384
NBLK = 20
R = M // NBLK
BPC = NBLK // NC
RPT = 312
RLAST = R - (NS - 1) * RPT
BPT = B // NS
NV = BPT // L
TRASH = R

SZS = (128, 64, 32, 16)
REG = {128: 0, 64: 128, 32: 192, 16: 224}
UPP = 8


def _sc_body(in_hbm, idx_hbm, val_hbm, out_hbm,
             my_idx, lb0, pb0, lb1, pb1,
             loc128, loc64, loc32, loc16, vbuf,
             blk0, blk1, gg, gs, si0, si1, so0, so1):
    c = lax.axis_index("c")
    s = lax.axis_index("s")

    blks = [blk0, blk1]
    sins = [si0, si1]
    souts = [so0, so1]
    lbs = [lb0, lb1]
    pbs = [pb0, pb1]
    locs = {128: loc128, 64: loc64, 32: loc32, 16: loc16}

    def blk_base(i):
        return (NC * i + c) * R

    def copy_in(i, sem, start):
        base = blk_base(i)
        buf = blks[i % 2]

        @pl.when(s < NS - 1)
        def _():
            cp = pltpu.make_async_copy(
                in_hbm.at[pl.ds(base + s * RPT, RPT)],
                buf.at[pl.ds(s * RPT, RPT)], sem)
            cp.start() if start else cp.wait()

        @pl.when(s == NS - 1)
        def _():
            cp = pltpu.make_async_copy(
                in_hbm.at[pl.ds(base + (NS - 1) * RPT, RLAST)],
                buf.at[pl.ds((NS - 1) * RPT, RLAST)], sem)
            cp.start() if start else cp.wait()

    def copy_out(i, sem, start):
        base = blk_base(i)
        buf = blks[i % 2]

        @pl.when(s < NS - 1)
        def _():
            cp = pltpu.make_async_copy(
                buf.at[pl.ds(s * RPT, RPT)],
                out_hbm.at[pl.ds(base + s * RPT, RPT)], sem)
            cp.start() if start else cp.wait()

        @pl.when(s == NS - 1)
        def _():
            cp = pltpu.make_async_copy(
                buf.at[pl.ds((NS - 1) * RPT, RLAST)],
                out_hbm.at[pl.ds(base + (NS - 1) * RPT, RLAST)], sem)
            cp.start() if start else cp.wait()

    def compact(i, lb, pb):
        base = blk_base(i)

        def cbody(j, cnt):
            iv = my_idx[pl.ds(j * L, L)]
            basev = jnp.full((L,), base, jnp.int32)
            limv = jnp.full((L,), base + R, jnp.int32)
            m = (iv >= basev) & (iv < limv)
            loc = iv - basev
            pos = jnp.full((L,), s * BPT + j * L, jnp.int32) + lax.iota(jnp.int32, L)
            mi = jnp.where(m, 1, 0).astype(jnp.int32)
            pc = plsc.cumsum(mi)
            dest = jnp.full((L,), cnt, jnp.int32) + pc - 1
            plsc.store_scatter(lb, [dest], loc, mask=m)
            plsc.store_scatter(pb, [dest], pos, mask=m)
            return cnt + jnp.sum(mi)

        cnt = lax.fori_loop(0, NV, cbody, jnp.int32(0))

        vstart = (cnt // L) * L
        lane = jnp.full((L,), vstart, jnp.int32) + lax.iota(jnp.int32, L)
        valid = lane < jnp.full((L,), cnt, jnp.int32)
        lv = lb[pl.ds(vstart, L)]
        pv = pb[pl.ds(vstart, L)]
        lb[pl.ds(vstart, L)] = jnp.where(valid, lv, TRASH)
        pb[pl.ds(vstart, L)] = jnp.where(valid, pv, 0)
        return cnt

    def decomp(units, start_u):
        u = jnp.minimum(units - start_u, UPP)
        offs = {}
        cur = start_u * L
        for S in SZS:
            su = S // L
            bit = (u // su) % 2 if su != UPP else u // su
            offs[S] = cur
            cur = cur + bit * S
        conds = {
            128: u >= 8,
            64: (u // 4) % 2 == 1,
            32: (u // 2) % 2 == 1,
            16: u % 2 == 1,
        }
        return offs, conds

    def fire_g(pb, offs, conds, lb):
        for S in SZS:
            @pl.when(conds[S])
            def _(S=S):
                pltpu.async_copy(
                    val_hbm.at[pb.at[pl.ds(offs[S], S)]],
                    vbuf.at[pl.ds(REG[S], S)], gg)
                for t in range(S // L):
                    locs[S][pl.ds(t * L, L)] = lb[pl.ds(offs[S] + t * L, L)]

    def drain_g(pb, offs, conds):
        for S in SZS:
            @pl.when(conds[S])
            def _(S=S):
                pltpu.make_async_copy(
                    val_hbm.at[pb.at[pl.ds(offs[S], S)]],
                    vbuf.at[pl.ds(REG[S], S)], gg).wait()

    def fire_s(blk, conds):
        for S in SZS:
            @pl.when(conds[S])
            def _(S=S):
                pltpu.async_copy(
                    vbuf.at[pl.ds(REG[S], S)],
                    blk.at[locs[S]], gs, add=True)

    def drain_s(blk, conds):
        for S in SZS:
            @pl.when(conds[S])
            def _(S=S):
                pltpu.make_async_copy(
                    vbuf.at[pl.ds(REG[S], S)],
                    blk.at[locs[S]], gs).wait()

    pltpu.sync_copy(idx_hbm.at[pl.ds(s * BPT, BPT)], my_idx)
    copy_in(0, sins[0], True)
    cnt = compact(0, lbs[0], pbs[0])
    units = (cnt + L - 1) // L
    offs, conds = decomp(units, 0)
    copy_in(0, sins[0], False)
    plsc.subcore_barrier()

    for i in range(BPC):
        q = i % 2
        blk = blks[q]
        npass = (units + UPP - 1) // UPP

        with jax.named_scope("dg"):
            fire_g(pbs[q], offs, conds, lbs[q])
            drain_g(pbs[q], offs, conds)
        fire_s(blk, conds)

        if i + 1 < BPC:
            if i >= 1:
                copy_out(i - 1, souts[1 - q], False)
            copy_in(i + 1, sins[1 - q], True)
            cnt1 = compact(i + 1, lbs[1 - q], pbs[1 - q])
            units1 = (cnt1 + L - 1) // L
            offs1, conds1 = decomp(units1, 0)

        with jax.named_scope("ds"):
            drain_s(blk, conds)

        def pass_body(ps, carry):
            offsx, condsx = decomp(units, ps * UPP)
            fire_g(pbs[q], offsx, condsx, lbs[q])
            drain_g(pbs[q], offsx, condsx)
            fire_s(blk, condsx)
            drain_s(blk, condsx)
            return carry

        lax.fori_loop(1, npass, pass_body, jnp.int32(0))

        if i + 1 < BPC:
            with jax.named_scope("cinw"):
                copy_in(i + 1, sins[1 - q], False)
            cnt, units, offs, conds = cnt1, units1, offs1, conds1

        with jax.named_scope("bar"):
            plsc.subcore_barrier()
        copy_out(i, souts[q], True)

    copy_out(BPC - 2, souts[(BPC - 2) % 2], False)
    copy_out(BPC - 1, souts[(BPC - 1) % 2], False)


@jax.jit
def _scatter_add(input, idx32, value):
    kfn = functools.partial(
        pl.kernel,
        mesh=plsc.VectorSubcoreMesh(core_axis_name="c", subcore_axis_name="s"),
        out_type=jax.ShapeDtypeStruct((M, D), jnp.float32),
        scratch_types=[
            pltpu.VMEM((BPT,), jnp.int32),
            pltpu.VMEM((BPT + 2 * L,), jnp.int32),
            pltpu.VMEM((BPT + 2 * L,), jnp.int32),
            pltpu.VMEM((BPT + 2 * L,), jnp.int32),
            pltpu.VMEM((BPT + 2 * L,), jnp.int32),
            pltpu.VMEM((128,), jnp.int32),
            pltpu.VMEM((64,), jnp.int32),
            pltpu.VMEM((32,), jnp.int32),
            pltpu.VMEM((16,), jnp.int32),
            pltpu.VMEM((256, D), jnp.float32),
            pltpu.VMEM_SHARED((R + L, D), jnp.float32),
            pltpu.VMEM_SHARED((R + L, D), jnp.float32),
            pltpu.SemaphoreType.DMA,
            pltpu.SemaphoreType.DMA,
            pltpu.SemaphoreType.DMA,
            pltpu.SemaphoreType.DMA,
            pltpu.SemaphoreType.DMA,
            pltpu.SemaphoreType.DMA,
        ],
        compiler_params=pltpu.CompilerParams(needs_layout_passes=False),
    )(_sc_body)
    return kfn(input, idx32, value)


def kernel(input, index, value):
    assert input.shape == (M, D) and value.shape == (B, D)
    return _scatter_add(input, index.astype(jnp.int32), value)
